# baseline (speedup 1.0000x reference)
import functools

import jax
import jax.numpy as jnp
from jax import lax
from jax.experimental import pallas as pl
from jax.experimental.pallas import tpu as pltpu
from jax.experimental.pallas import tpu_sc as plsc

_D, _H, _W = 32, 128, 128
_R = _D * _H
_N = _D * _H * _W
_KP = 64
_MINC = 50.0
_NC, _NS = 2, 16
_NWORK = _NC * _NS
_RPW = _R // _NWORK
_EPW = _N // _NWORK
_SLC = _N // _NS


def _vote_idx_body(ez_ref, ey_ref, ex_ref, fg_ref, idx_ref, fgf_ref):
    zi = jnp.clip(jnp.floor(ez_ref[...] * 0.5).astype(jnp.int32), 0, _D - 1)
    yi = jnp.clip(jnp.floor(ey_ref[...]).astype(jnp.int32), 0, _H - 1)
    xi = jnp.clip(jnp.floor(ex_ref[...]).astype(jnp.int32), 0, _W - 1)
    idx_ref[...] = (zi * _H + yi) * _W + xi
    fgf_ref[...] = fg_ref[...].astype(jnp.float32)


def _vote_idx(ez, ey, ex, fg, interpret=False):
    blk = pl.BlockSpec((512, _W), lambda i: (i, 0))
    return pl.pallas_call(
        _vote_idx_body,
        grid=(8,),
        in_specs=[blk] * 4,
        out_specs=[blk, blk],
        out_shape=[
            jax.ShapeDtypeStruct((_R, _W), jnp.int32),
            jax.ShapeDtypeStruct((_R, _W), jnp.float32),
        ],
        interpret=interpret,
    )(ez, ey, ex, fg)


def _sc_scatter_body(idx_hbm, val_hbm, out_hbm, idx_v, val_v, zb,
                     votes_sh, sem):
    c = lax.axis_index("c")
    s = lax.axis_index("s")
    wid = c * _NS + s

    el0 = wid * _EPW
    in0 = pltpu.async_copy(idx_hbm.at[pl.ds(el0, _EPW)], idx_v, sem)
    in1 = pltpu.async_copy(val_hbm.at[pl.ds(el0, _EPW)], val_v, sem)

    zv = jnp.zeros((16,), jnp.float32)

    def zbody(i, carry):
        base = i * 512
        for u in range(32):
            zb[pl.ds(base + u * 16, 16)] = zv
        return carry

    lax.fori_loop(0, _SLC // 512, zbody, 0)

    base = s * _SLC
    pltpu.sync_copy(zb, votes_sh.at[pl.ds(base, _SLC)])
    in0.wait()
    in1.wait()
    plsc.subcore_barrier()

    pltpu.sync_copy(val_v, votes_sh.at[idx_v], add=True)
    plsc.subcore_barrier()

    pltpu.sync_copy(votes_sh.at[pl.ds(base, _SLC)], zb)
    pltpu.sync_copy(zb, out_hbm.at[c, pl.ds(base, _SLC)])


@functools.cache
def _sc_scatter():
    return functools.partial(
        pl.kernel,
        mesh=plsc.VectorSubcoreMesh(core_axis_name="c", subcore_axis_name="s"),
        out_type=jax.ShapeDtypeStruct((_NC, _N), jnp.float32),
        scratch_types=[
            pltpu.VMEM((_EPW,), jnp.int32),
            pltpu.VMEM((_EPW,), jnp.float32),
            pltpu.VMEM((_SLC,), jnp.float32),
            pltpu.VMEM_SHARED((_N,), jnp.float32),
            pltpu.SemaphoreType.DMA,
        ],
    )(_sc_scatter_body)


def _shl_lanes(x, s):
    r, w = x.shape
    return jnp.concatenate([x[:, s:], jnp.zeros((r, s), x.dtype)], axis=1)


def _shr_lanes(x, s):
    r, w = x.shape
    return jnp.concatenate([jnp.zeros((r, s), x.dtype), x[:, : w - s]], axis=1)


def _shl_rows(x, s):
    r, w = x.shape
    return jnp.concatenate([x[s:, :], jnp.zeros((s, w), x.dtype)], axis=0)


def _shr_rows(x, s):
    r, w = x.shape
    return jnp.concatenate([jnp.zeros((s, w), x.dtype), x[: r - s, :]], axis=0)


def _win11(x, shl, shr):
    m2 = jnp.maximum(x, shl(x, 1))
    m3 = jnp.maximum(m2, shl(x, 2))
    m4 = jnp.maximum(m2, shl(m2, 2))
    m8 = jnp.maximum(m4, shl(m4, 4))
    m11 = jnp.maximum(m8, shl(m3, 8))
    return shr(m11, 5)


def _peaks_body(pv_ref, pos_ref, nv_ref, votes_ref, t0_ref, t1_ref,
                gmax_ref, vals_ref, flats_ref):
    for b in range(8):
        sl = pl.ds(b * 512, 512)
        v = pv_ref[0, sl, :] + pv_ref[1, sl, :]
        votes_ref[sl, :] = v
        t0_ref[sl, :] = _win11(v, _shl_lanes, _shr_lanes)

    for d in range(_D):
        sl = pl.ds(d * _H, _H)
        t1_ref[sl, :] = _win11(t0_ref[sl, :], _shl_rows, _shr_rows)

    for d in range(_D):
        m = t1_ref[pl.ds(d * _H, _H), :]
        for o in range(-3, 4):
            dd = d + o
            if 0 <= dd < _D and o != 0:
                m = jnp.maximum(m, t1_ref[pl.ds(dd * _H, _H), :])
        v = votes_ref[pl.ds(d * _H, _H), :]
        sc = jnp.where((v >= m) & (v >= _MINC), v, 0.0)
        t0_ref[pl.ds(d * _H, _H), :] = sc
        gmax_ref[pl.ds(d * 2, 2), :] = jnp.max(sc.reshape(2, 64, _W), axis=1)

    vals_ref[...] = jnp.zeros((1, 128), jnp.float32)
    flats_ref[...] = jnp.zeros((1, 128), jnp.int32)

    g_iota = lax.broadcasted_iota(jnp.int32, (64, 128), 0)
    r_iota = lax.broadcasted_iota(jnp.int32, (64, 128), 0)
    w_iota = lax.broadcasted_iota(jnp.int32, (64, 128), 1)
    l_iota = lax.broadcasted_iota(jnp.int32, (1, 128), 1)

    @pl.when(jnp.max(gmax_ref[...]) >= _MINC)
    def _():
        def tk_body(k, carry):
            gm = gmax_ref[...]
            m = jnp.max(gm)

            @pl.when(m >= _MINC)
            def _():
                g = jnp.min(jnp.where(gm == m, g_iota, _N))
                slab = t0_ref[pl.ds(g * 64, 64), :]
                rank = jnp.min(jnp.where(slab == m, r_iota * _W + w_iota, _N))
                flat = g * (64 * _W) + rank
                newslab = jnp.where(r_iota * _W + w_iota == rank, -1.0, slab)
                t0_ref[pl.ds(g * 64, 64), :] = newslab
                gmax_ref[pl.ds(g, 1), :] = jnp.max(newslab.reshape(1, 64, _W),
                                                   axis=1)
                vals_ref[...] = jnp.where(l_iota == k, m, vals_ref[...])
                flats_ref[...] = jnp.where(l_iota == k, flat, flats_ref[...])

            return carry

        lax.fori_loop(0, _KP, tk_body, 0)

    vals = vals_ref[...]
    flats = flats_ref[...]
    z = flats >> 14
    y = (flats >> 7) & (_H - 1)
    x = flats & (_W - 1)
    validf = jnp.where(vals >= _MINC, 1.0, 0.0)
    pos_ref[pl.ds(0, 1), :] = (z.astype(jnp.float32) + 0.5) * 2.0
    pos_ref[pl.ds(1, 1), :] = y.astype(jnp.float32) + 0.5
    pos_ref[pl.ds(2, 1), :] = x.astype(jnp.float32) + 0.5
    pos_ref[pl.ds(3, 1), :] = validf
    pos_ref[pl.ds(4, 4), :] = jnp.zeros((4, 128), jnp.float32)
    nv_ref[...] = jnp.full((1, 1), jnp.sum(validf).astype(jnp.int32), jnp.int32)


def _peaks(partials, interpret=False):
    return pl.pallas_call(
        _peaks_body,
        out_shape=[
            jax.ShapeDtypeStruct((8, 128), jnp.float32),
            jax.ShapeDtypeStruct((1, 1), jnp.int32),
        ],
        scratch_shapes=[
            pltpu.VMEM((_R, _W), jnp.float32),
            pltpu.VMEM((_R, _W), jnp.float32),
            pltpu.VMEM((_R, _W), jnp.float32),
            pltpu.VMEM((64, 128), jnp.float32),
            pltpu.VMEM((1, 128), jnp.float32),
            pltpu.VMEM((1, 128), jnp.int32),
        ],
        interpret=interpret,
    )(partials)


def _labels_body(pos_ref, nv_ref, ez_hbm, ey_hbm, ex_hbm, fgf_hbm, out_ref,
                 ez_s, ey_s, ex_s, fgf_s, sem):
    nv = nv_ref[0, 0]

    @pl.when(nv == 0)
    def _():
        for b in range(16):
            out_ref[pl.ds(b * 256, 256), :] = jnp.zeros((256, _W), jnp.int32)

    @pl.when(nv > 0)
    def _():
        cps = [pltpu.make_async_copy(h, s, sem)
               for h, s in ((ez_hbm, ez_s), (ey_hbm, ey_s),
                            (ex_hbm, ex_s), (fgf_hbm, fgf_s))]
        for cp in cps:
            cp.start()
        for cp in cps:
            cp.wait()
        inf = jnp.float32(jnp.inf)
        for b in range(16):
            sl = pl.ds(b * 256, 256)
            ez = ez_s[sl, :]
            ey = ey_s[sl, :]
            ex = ex_s[sl, :]
            e2 = (ez * ez + ey * ey) + ex * ex

            def body(k, carry):
                bd, bi = carry
                pz = pos_ref[0, k]
                py = pos_ref[1, k]
                px = pos_ref[2, k]
                vk = pos_ref[3, k]
                p2 = (pz * pz + py * py) + px * px
                dot = (ez * pz + ey * py) + ex * px
                d = (e2 + p2) - 2.0 * dot
                d = jnp.where(vk > 0.0, d, inf)
                better = d < bd
                return jnp.where(better, d, bd), jnp.where(better, k, bi)

            bd0 = jnp.full((256, _W), inf, jnp.float32)
            bi0 = jnp.zeros((256, _W), jnp.int32)
            bd, bi = lax.fori_loop(0, _KP, body, (bd0, bi0))
            out_ref[sl, :] = jnp.where(fgf_s[sl, :] > 0.0, bi + 1, 0)


def _labels(pos, nv, ez, ey, ex, fgf, interpret=False):
    return pl.pallas_call(
        _labels_body,
        in_specs=[
            pl.BlockSpec(memory_space=pltpu.SMEM),
            pl.BlockSpec(memory_space=pltpu.SMEM),
            pl.BlockSpec(memory_space=pltpu.HBM),
            pl.BlockSpec(memory_space=pltpu.HBM),
            pl.BlockSpec(memory_space=pltpu.HBM),
            pl.BlockSpec(memory_space=pltpu.HBM),
        ],
        out_shape=jax.ShapeDtypeStruct((_R, _W), jnp.int32),
        scratch_shapes=[
            pltpu.VMEM((_R, _W), jnp.float32),
            pltpu.VMEM((_R, _W), jnp.float32),
            pltpu.VMEM((_R, _W), jnp.float32),
            pltpu.VMEM((_R, _W), jnp.float32),
            pltpu.SemaphoreType.DMA,
        ],
        interpret=interpret,
    )(pos, nv, ez, ey, ex, fgf)


def kernel(embeddings, fg_mask):
    ez = embeddings[..., 0].reshape(_R, _W)
    ey = embeddings[..., 1].reshape(_R, _W)
    ex = embeddings[..., 2].reshape(_R, _W)
    fg = fg_mask.reshape(_R, _W)

    idx, fgf = _vote_idx(ez, ey, ex, fg)
    partials = _sc_scatter()(idx.reshape(_N), fgf.reshape(_N))
    pos, nv = _peaks(partials.reshape(_NC, _R, _W))
    labels = _labels(pos, nv, ez, ey, ex, fgf)
    return labels.reshape(_D, _H, _W)


# speedup vs baseline: 1.0574x; 1.0574x over previous
import functools

import jax
import jax.numpy as jnp
from jax import lax
from jax.experimental import pallas as pl
from jax.experimental.pallas import tpu as pltpu
from jax.experimental.pallas import tpu_sc as plsc

_D, _H, _W = 32, 128, 128
_R = _D * _H
_N = _D * _H * _W
_KP = 64
_MINC = 50.0
_NC, _NS = 2, 16
_NWORK = _NC * _NS
_RPW = _R // _NWORK
_EPW = _N // _NWORK
_SLC = _N // _NS


def _vote_idx_body(ez_ref, ey_ref, ex_ref, idx_ref):
    zi = jnp.clip(jnp.floor(ez_ref[...] * 0.5).astype(jnp.int32), 0, _D - 1)
    yi = jnp.clip(jnp.floor(ey_ref[...]).astype(jnp.int32), 0, _H - 1)
    xi = jnp.clip(jnp.floor(ex_ref[...]).astype(jnp.int32), 0, _W - 1)
    idx_ref[...] = (zi * _H + yi) * _W + xi


def _vote_idx(ez, ey, ex, interpret=False):
    blk = pl.BlockSpec((1024, _W), lambda i: (i, 0))
    return pl.pallas_call(
        _vote_idx_body,
        grid=(4,),
        in_specs=[blk] * 3,
        out_specs=blk,
        out_shape=jax.ShapeDtypeStruct((_R, _W), jnp.int32),
        interpret=interpret,
    )(ez, ey, ex)


def _sc_scatter_body(idx_hbm, val_hbm, out_hbm, idx_v, val_v, zb,
                     votes_sh, sem):
    c = lax.axis_index("c")
    s = lax.axis_index("s")
    wid = c * _NS + s

    el0 = wid * _EPW
    in0 = pltpu.async_copy(idx_hbm.at[pl.ds(el0, _EPW)], idx_v, sem)
    in1 = pltpu.async_copy(val_hbm.at[pl.ds(el0, _EPW)], val_v, sem)

    zv = jnp.zeros((16,), jnp.float32)

    def zbody(i, carry):
        base = i * 512
        for u in range(32):
            zb[pl.ds(base + u * 16, 16)] = zv
        return carry

    lax.fori_loop(0, _SLC // 512, zbody, 0)

    base = s * _SLC
    pltpu.sync_copy(zb, votes_sh.at[pl.ds(base, _SLC)])
    in0.wait()
    in1.wait()
    plsc.subcore_barrier()

    pltpu.sync_copy(val_v, votes_sh.at[idx_v], add=True)
    plsc.subcore_barrier()

    pltpu.sync_copy(votes_sh.at[pl.ds(base, _SLC)], zb)
    pltpu.sync_copy(zb, out_hbm.at[c, pl.ds(base, _SLC)])


@functools.cache
def _sc_scatter():
    return functools.partial(
        pl.kernel,
        mesh=plsc.VectorSubcoreMesh(core_axis_name="c", subcore_axis_name="s"),
        out_type=jax.ShapeDtypeStruct((_NC, _N), jnp.float32),
        scratch_types=[
            pltpu.VMEM((_EPW,), jnp.int32),
            pltpu.VMEM((_EPW,), jnp.float32),
            pltpu.VMEM((_SLC,), jnp.float32),
            pltpu.VMEM_SHARED((_N,), jnp.float32),
            pltpu.SemaphoreType.DMA,
        ],
    )(_sc_scatter_body)


def _shl_lanes(x, s):
    r, w = x.shape
    return jnp.concatenate([x[:, s:], jnp.zeros((r, s), x.dtype)], axis=1)


def _shr_lanes(x, s):
    r, w = x.shape
    return jnp.concatenate([jnp.zeros((r, s), x.dtype), x[:, : w - s]], axis=1)


def _shl_rows(x, s):
    r, w = x.shape
    return jnp.concatenate([x[s:, :], jnp.zeros((s, w), x.dtype)], axis=0)


def _shr_rows(x, s):
    r, w = x.shape
    return jnp.concatenate([jnp.zeros((s, w), x.dtype), x[: r - s, :]], axis=0)


def _win11(x, shl, shr):
    m2 = jnp.maximum(x, shl(x, 1))
    m3 = jnp.maximum(m2, shl(x, 2))
    m4 = jnp.maximum(m2, shl(m2, 2))
    m8 = jnp.maximum(m4, shl(m4, 4))
    m11 = jnp.maximum(m8, shl(m3, 8))
    return shr(m11, 5)


def _peaks_body(pv_ref, pos_ref, nv_ref, votes_ref, t0_ref, t1_ref,
                gmax_ref, vals_ref, flats_ref):
    for b in range(8):
        sl = pl.ds(b * 512, 512)
        v = pv_ref[0, sl, :] + pv_ref[1, sl, :]
        votes_ref[sl, :] = v
        t0_ref[sl, :] = _win11(v, _shl_lanes, _shr_lanes)

    for d in range(_D):
        sl = pl.ds(d * _H, _H)
        t1_ref[sl, :] = _win11(t0_ref[sl, :], _shl_rows, _shr_rows)

    for d in range(_D):
        m = t1_ref[pl.ds(d * _H, _H), :]
        for o in range(-3, 4):
            dd = d + o
            if 0 <= dd < _D and o != 0:
                m = jnp.maximum(m, t1_ref[pl.ds(dd * _H, _H), :])
        v = votes_ref[pl.ds(d * _H, _H), :]
        sc = jnp.where((v >= m) & (v >= _MINC), v, 0.0)
        t0_ref[pl.ds(d * _H, _H), :] = sc
        gmax_ref[pl.ds(d * 2, 2), :] = jnp.max(sc.reshape(2, 64, _W), axis=1)

    vals_ref[...] = jnp.zeros((1, 128), jnp.float32)
    flats_ref[...] = jnp.zeros((1, 128), jnp.int32)

    g_iota = lax.broadcasted_iota(jnp.int32, (64, 128), 0)
    r_iota = lax.broadcasted_iota(jnp.int32, (64, 128), 0)
    w_iota = lax.broadcasted_iota(jnp.int32, (64, 128), 1)
    l_iota = lax.broadcasted_iota(jnp.int32, (1, 128), 1)

    @pl.when(jnp.max(gmax_ref[...]) >= _MINC)
    def _():
        def tk_body(k, carry):
            gm = gmax_ref[...]
            m = jnp.max(gm)

            @pl.when(m >= _MINC)
            def _():
                g = jnp.min(jnp.where(gm == m, g_iota, _N))
                slab = t0_ref[pl.ds(g * 64, 64), :]
                rank = jnp.min(jnp.where(slab == m, r_iota * _W + w_iota, _N))
                flat = g * (64 * _W) + rank
                newslab = jnp.where(r_iota * _W + w_iota == rank, -1.0, slab)
                t0_ref[pl.ds(g * 64, 64), :] = newslab
                gmax_ref[pl.ds(g, 1), :] = jnp.max(newslab.reshape(1, 64, _W),
                                                   axis=1)
                vals_ref[...] = jnp.where(l_iota == k, m, vals_ref[...])
                flats_ref[...] = jnp.where(l_iota == k, flat, flats_ref[...])

            return carry

        lax.fori_loop(0, _KP, tk_body, 0)

    vals = vals_ref[...]
    flats = flats_ref[...]
    z = flats >> 14
    y = (flats >> 7) & (_H - 1)
    x = flats & (_W - 1)
    validf = jnp.where(vals >= _MINC, 1.0, 0.0)
    pos_ref[pl.ds(0, 1), :] = (z.astype(jnp.float32) + 0.5) * 2.0
    pos_ref[pl.ds(1, 1), :] = y.astype(jnp.float32) + 0.5
    pos_ref[pl.ds(2, 1), :] = x.astype(jnp.float32) + 0.5
    pos_ref[pl.ds(3, 1), :] = validf
    pos_ref[pl.ds(4, 4), :] = jnp.zeros((4, 128), jnp.float32)
    nv_ref[...] = jnp.full((1, 1), jnp.sum(validf).astype(jnp.int32), jnp.int32)


def _peaks(partials, interpret=False):
    return pl.pallas_call(
        _peaks_body,
        out_shape=[
            jax.ShapeDtypeStruct((8, 128), jnp.float32),
            jax.ShapeDtypeStruct((1, 1), jnp.int32),
        ],
        scratch_shapes=[
            pltpu.VMEM((_R, _W), jnp.float32),
            pltpu.VMEM((_R, _W), jnp.float32),
            pltpu.VMEM((_R, _W), jnp.float32),
            pltpu.VMEM((64, 128), jnp.float32),
            pltpu.VMEM((1, 128), jnp.float32),
            pltpu.VMEM((1, 128), jnp.int32),
        ],
        interpret=interpret,
    )(partials)


def _labels_body(pos_ref, nv_ref, ez_hbm, ey_hbm, ex_hbm, fgf_hbm, out_ref,
                 ez_s, ey_s, ex_s, fgf_s, sem):
    nv = nv_ref[0, 0]

    @pl.when(nv == 0)
    def _():
        for b in range(16):
            out_ref[pl.ds(b * 256, 256), :] = jnp.zeros((256, _W), jnp.int32)

    @pl.when(nv > 0)
    def _():
        cps = [pltpu.make_async_copy(h, s, sem)
               for h, s in ((ez_hbm, ez_s), (ey_hbm, ey_s),
                            (ex_hbm, ex_s), (fgf_hbm, fgf_s))]
        for cp in cps:
            cp.start()
        for cp in cps:
            cp.wait()
        inf = jnp.float32(jnp.inf)
        for b in range(16):
            sl = pl.ds(b * 256, 256)
            ez = ez_s[sl, :]
            ey = ey_s[sl, :]
            ex = ex_s[sl, :]
            e2 = (ez * ez + ey * ey) + ex * ex

            def body(k, carry):
                bd, bi = carry
                pz = pos_ref[0, k]
                py = pos_ref[1, k]
                px = pos_ref[2, k]
                vk = pos_ref[3, k]
                p2 = (pz * pz + py * py) + px * px
                dot = (ez * pz + ey * py) + ex * px
                d = (e2 + p2) - 2.0 * dot
                d = jnp.where(vk > 0.0, d, inf)
                better = d < bd
                return jnp.where(better, d, bd), jnp.where(better, k, bi)

            bd0 = jnp.full((256, _W), inf, jnp.float32)
            bi0 = jnp.zeros((256, _W), jnp.int32)
            bd, bi = lax.fori_loop(0, _KP, body, (bd0, bi0))
            out_ref[sl, :] = jnp.where(fgf_s[sl, :] > 0.0, bi + 1, 0)


def _labels(pos, nv, ez, ey, ex, fgf, interpret=False):
    return pl.pallas_call(
        _labels_body,
        in_specs=[
            pl.BlockSpec(memory_space=pltpu.SMEM),
            pl.BlockSpec(memory_space=pltpu.SMEM),
            pl.BlockSpec(memory_space=pltpu.HBM),
            pl.BlockSpec(memory_space=pltpu.HBM),
            pl.BlockSpec(memory_space=pltpu.HBM),
            pl.BlockSpec(memory_space=pltpu.HBM),
        ],
        out_shape=jax.ShapeDtypeStruct((_R, _W), jnp.int32),
        scratch_shapes=[
            pltpu.VMEM((_R, _W), jnp.float32),
            pltpu.VMEM((_R, _W), jnp.float32),
            pltpu.VMEM((_R, _W), jnp.float32),
            pltpu.VMEM((_R, _W), jnp.float32),
            pltpu.SemaphoreType.DMA,
        ],
        interpret=interpret,
    )(pos, nv, ez, ey, ex, fgf)


def kernel(embeddings, fg_mask):
    ez = embeddings[..., 0].reshape(_R, _W)
    ey = embeddings[..., 1].reshape(_R, _W)
    ex = embeddings[..., 2].reshape(_R, _W)
    fgf = fg_mask.reshape(_R, _W).astype(jnp.float32)

    idx = _vote_idx(ez, ey, ex)
    partials = _sc_scatter()(idx.reshape(_N), fgf.reshape(_N))
    pos, nv = _peaks(partials.reshape(_NC, _R, _W))
    labels = _labels(pos, nv, ez, ey, ex, fgf)
    return labels.reshape(_D, _H, _W)


# speedup vs baseline: 1.0946x; 1.0352x over previous
import functools

import jax
import jax.numpy as jnp
from jax import lax
from jax.experimental import pallas as pl
from jax.experimental.pallas import tpu as pltpu
from jax.experimental.pallas import tpu_sc as plsc

_D, _H, _W = 32, 128, 128
_R = _D * _H
_N = _D * _H * _W
_KP = 64
_MINC = 50.0
_NC, _NS = 2, 16
_NWORK = _NC * _NS
_RPW = _R // _NWORK
_EPW = _N // _NWORK
_SLC = _N // _NS


def _vote_idx_body(ez_ref, ey_ref, ex_ref, idx_ref):
    zi = jnp.clip(jnp.floor(ez_ref[...] * 0.5).astype(jnp.int32), 0, _D - 1)
    yi = jnp.clip(jnp.floor(ey_ref[...]).astype(jnp.int32), 0, _H - 1)
    xi = jnp.clip(jnp.floor(ex_ref[...]).astype(jnp.int32), 0, _W - 1)
    idx_ref[...] = (zi * _H + yi) * _W + xi


def _vote_idx(ez, ey, ex, interpret=False):
    blk = pl.BlockSpec((1024, _W), lambda i: (i, 0))
    return pl.pallas_call(
        _vote_idx_body,
        grid=(4,),
        in_specs=[blk] * 3,
        out_specs=blk,
        out_shape=jax.ShapeDtypeStruct((_R, _W), jnp.int32),
        interpret=interpret,
    )(ez, ey, ex)


def _sc_scatter_body(idx_hbm, val_hbm, out_hbm, idx_v, val_v, zb,
                     votes_sh, sem):
    c = lax.axis_index("c")
    s = lax.axis_index("s")
    wid = c * _NS + s

    el0 = wid * _EPW
    in0 = pltpu.async_copy(idx_hbm.at[pl.ds(el0, _EPW)], idx_v, sem)
    in1 = pltpu.async_copy(val_hbm.at[pl.ds(el0, _EPW)], val_v, sem)

    zv = jnp.zeros((16,), jnp.float32)

    def zbody(i, carry):
        base = i * 512
        for u in range(32):
            zb[pl.ds(base + u * 16, 16)] = zv
        return carry

    lax.fori_loop(0, _SLC // 512, zbody, 0)

    base = s * _SLC
    pltpu.sync_copy(zb, votes_sh.at[pl.ds(base, _SLC)])
    in0.wait()
    in1.wait()
    plsc.subcore_barrier()

    pltpu.sync_copy(val_v, votes_sh.at[idx_v], add=True)
    plsc.subcore_barrier()

    pltpu.sync_copy(votes_sh.at[pl.ds(base, _SLC)], zb)
    pltpu.sync_copy(zb, out_hbm.at[c, pl.ds(base, _SLC)])


@functools.cache
def _sc_scatter():
    return functools.partial(
        pl.kernel,
        mesh=plsc.VectorSubcoreMesh(core_axis_name="c", subcore_axis_name="s"),
        out_type=jax.ShapeDtypeStruct((_NC, _N), jnp.float32),
        scratch_types=[
            pltpu.VMEM((_EPW,), jnp.int32),
            pltpu.VMEM((_EPW,), jnp.float32),
            pltpu.VMEM((_SLC,), jnp.float32),
            pltpu.VMEM_SHARED((_N,), jnp.float32),
            pltpu.SemaphoreType.DMA,
        ],
    )(_sc_scatter_body)


def _shl_lanes(x, s):
    r, w = x.shape
    return jnp.concatenate([x[:, s:], jnp.zeros((r, s), x.dtype)], axis=1)


def _shr_lanes(x, s):
    r, w = x.shape
    return jnp.concatenate([jnp.zeros((r, s), x.dtype), x[:, : w - s]], axis=1)


def _shl_rows(x, s):
    r, w = x.shape
    return jnp.concatenate([x[s:, :], jnp.zeros((s, w), x.dtype)], axis=0)


def _shr_rows(x, s):
    r, w = x.shape
    return jnp.concatenate([jnp.zeros((s, w), x.dtype), x[: r - s, :]], axis=0)


def _win11(x, shl, shr):
    m2 = jnp.maximum(x, shl(x, 1))
    m3 = jnp.maximum(m2, shl(x, 2))
    m4 = jnp.maximum(m2, shl(m2, 2))
    m8 = jnp.maximum(m4, shl(m4, 4))
    m11 = jnp.maximum(m8, shl(m3, 8))
    return shr(m11, 5)


def _peaks_body(pv_ref, ez_hbm, ey_hbm, ex_hbm, fgf_hbm, out_ref,
                votes_ref, t0_ref, t1_ref, gmax_ref, vals_ref, flats_ref,
                posv_ref, pos_sm, ez_s, ey_s, ex_s, fgf_s, sem):
    for b in range(8):
        sl = pl.ds(b * 512, 512)
        v = pv_ref[0, sl, :] + pv_ref[1, sl, :]
        votes_ref[sl, :] = v
        t0_ref[sl, :] = _win11(v, _shl_lanes, _shr_lanes)

    for d in range(_D):
        sl = pl.ds(d * _H, _H)
        t1_ref[sl, :] = _win11(t0_ref[sl, :], _shl_rows, _shr_rows)

    for d in range(_D):
        m = t1_ref[pl.ds(d * _H, _H), :]
        for o in range(-3, 4):
            dd = d + o
            if 0 <= dd < _D and o != 0:
                m = jnp.maximum(m, t1_ref[pl.ds(dd * _H, _H), :])
        v = votes_ref[pl.ds(d * _H, _H), :]
        sc = jnp.where((v >= m) & (v >= _MINC), v, 0.0)
        t0_ref[pl.ds(d * _H, _H), :] = sc
        gmax_ref[pl.ds(d * 2, 2), :] = jnp.max(sc.reshape(2, 64, _W), axis=1)

    vals_ref[...] = jnp.zeros((1, 128), jnp.float32)
    flats_ref[...] = jnp.zeros((1, 128), jnp.int32)

    g_iota = lax.broadcasted_iota(jnp.int32, (64, 128), 0)
    r_iota = lax.broadcasted_iota(jnp.int32, (64, 128), 0)
    w_iota = lax.broadcasted_iota(jnp.int32, (64, 128), 1)
    l_iota = lax.broadcasted_iota(jnp.int32, (1, 128), 1)

    @pl.when(jnp.max(gmax_ref[...]) >= _MINC)
    def _():
        def tk_body(k, carry):
            gm = gmax_ref[...]
            m = jnp.max(gm)

            @pl.when(m >= _MINC)
            def _():
                g = jnp.min(jnp.where(gm == m, g_iota, _N))
                slab = t0_ref[pl.ds(g * 64, 64), :]
                rank = jnp.min(jnp.where(slab == m, r_iota * _W + w_iota, _N))
                flat = g * (64 * _W) + rank
                newslab = jnp.where(r_iota * _W + w_iota == rank, -1.0, slab)
                t0_ref[pl.ds(g * 64, 64), :] = newslab
                gmax_ref[pl.ds(g, 1), :] = jnp.max(newslab.reshape(1, 64, _W),
                                                   axis=1)
                vals_ref[...] = jnp.where(l_iota == k, m, vals_ref[...])
                flats_ref[...] = jnp.where(l_iota == k, flat, flats_ref[...])

            return carry

        lax.fori_loop(0, _KP, tk_body, 0)

    vals = vals_ref[...]
    flats = flats_ref[...]
    z = flats >> 14
    y = (flats >> 7) & (_H - 1)
    x = flats & (_W - 1)
    validf = jnp.where(vals >= _MINC, 1.0, 0.0)
    nv = jnp.sum(validf).astype(jnp.int32)

    @pl.when(nv == 0)
    def _():
        for b in range(16):
            out_ref[pl.ds(b * 256, 256), :] = jnp.zeros((256, _W), jnp.int32)

    @pl.when(nv > 0)
    def _():
        posv_ref[pl.ds(0, 1), :] = (z.astype(jnp.float32) + 0.5) * 2.0
        posv_ref[pl.ds(1, 1), :] = y.astype(jnp.float32) + 0.5
        posv_ref[pl.ds(2, 1), :] = x.astype(jnp.float32) + 0.5
        posv_ref[pl.ds(3, 1), :] = validf
        posv_ref[pl.ds(4, 4), :] = jnp.zeros((4, 128), jnp.float32)
        cps = [pltpu.make_async_copy(h, s, sem)
               for h, s in ((ez_hbm, ez_s), (ey_hbm, ey_s),
                            (ex_hbm, ex_s), (fgf_hbm, fgf_s),
                            (posv_ref, pos_sm))]
        for cp in cps:
            cp.start()
        for cp in cps:
            cp.wait()
        inf = jnp.float32(jnp.inf)
        for b in range(16):
            sl = pl.ds(b * 256, 256)
            ez = ez_s[sl, :]
            ey = ey_s[sl, :]
            ex = ex_s[sl, :]
            e2 = (ez * ez + ey * ey) + ex * ex

            def body(k, carry):
                bd, bi = carry
                pz = pos_sm[0, k]
                py = pos_sm[1, k]
                px = pos_sm[2, k]
                vk = pos_sm[3, k]
                p2 = (pz * pz + py * py) + px * px
                dot = (ez * pz + ey * py) + ex * px
                d = (e2 + p2) - 2.0 * dot
                d = jnp.where(vk > 0.0, d, inf)
                better = d < bd
                return jnp.where(better, d, bd), jnp.where(better, k, bi)

            bd0 = jnp.full((256, _W), inf, jnp.float32)
            bi0 = jnp.zeros((256, _W), jnp.int32)
            bd, bi = lax.fori_loop(0, _KP, body, (bd0, bi0))
            out_ref[sl, :] = jnp.where(fgf_s[sl, :] > 0.0, bi + 1, 0)


def _peaks(partials, ez, ey, ex, fgf, interpret=False):
    return pl.pallas_call(
        _peaks_body,
        in_specs=[
            pl.BlockSpec((_NC, _R, _W), lambda: (0, 0, 0)),
            pl.BlockSpec(memory_space=pltpu.HBM),
            pl.BlockSpec(memory_space=pltpu.HBM),
            pl.BlockSpec(memory_space=pltpu.HBM),
            pl.BlockSpec(memory_space=pltpu.HBM),
        ],
        out_shape=jax.ShapeDtypeStruct((_R, _W), jnp.int32),
        scratch_shapes=[
            pltpu.VMEM((_R, _W), jnp.float32),
            pltpu.VMEM((_R, _W), jnp.float32),
            pltpu.VMEM((_R, _W), jnp.float32),
            pltpu.VMEM((64, 128), jnp.float32),
            pltpu.VMEM((1, 128), jnp.float32),
            pltpu.VMEM((1, 128), jnp.int32),
            pltpu.VMEM((8, 128), jnp.float32),
            pltpu.SMEM((8, 128), jnp.float32),
            pltpu.VMEM((_R, _W), jnp.float32),
            pltpu.VMEM((_R, _W), jnp.float32),
            pltpu.VMEM((_R, _W), jnp.float32),
            pltpu.VMEM((_R, _W), jnp.float32),
            pltpu.SemaphoreType.DMA,
        ],
        interpret=interpret,
    )(partials, ez, ey, ex, fgf)


def kernel(embeddings, fg_mask):
    ez = embeddings[..., 0].reshape(_R, _W)
    ey = embeddings[..., 1].reshape(_R, _W)
    ex = embeddings[..., 2].reshape(_R, _W)
    fgf = fg_mask.reshape(_R, _W).astype(jnp.float32)

    idx = _vote_idx(ez, ey, ex)
    partials = _sc_scatter()(idx.reshape(_N), fgf.reshape(_N))
    labels = _peaks(partials.reshape(_NC, _R, _W), ez, ey, ex, fgf)
    return labels.reshape(_D, _H, _W)
